# baseline (speedup 1.0000x reference)
import functools

import jax
import jax.numpy as jnp
from jax import lax
from jax.experimental import pallas as pl
from jax.experimental.pallas import tpu as pltpu
from jax.experimental.pallas import tpu_sc as plsc

N = 10000
E = 320000
NUM_REL = 4
NPAD = 10240
M = NUM_REL * NPAD
K = 128
ROWS_PER_TILE = M // 16
NB_BASE = (E // K) // 16
NB_EXTRA = (E // K) % 16
NBMAX = NB_BASE + 1
EPAD = NBMAX * K * 16
DEPTH = 5


def _make_segsum(nq: int, do_deg: bool):
  per_sc = nq // 2
  mesh = plsc.VectorSubcoreMesh(core_axis_name="c", subcore_axis_name="s")
  out_type = [jax.ShapeDtypeStruct((M, nq * 32), jnp.float32)]
  if do_deg:
    out_type.append(jax.ShapeDtypeStruct((32, M // 2), jnp.float32))
  scratch = [
      pltpu.VMEM((DEPTH, K, 32), jnp.float32),
      pltpu.VMEM((2, DEPTH, 3, K), jnp.int32),
      pltpu.VMEM((2, DEPTH, K), jnp.int32),
      pltpu.VMEM((DEPTH, K), jnp.int32),
  ]
  if do_deg:
    scratch += [pltpu.VMEM((M // 2,), jnp.float32)]
  scratch += [pltpu.VMEM_SHARED((M, 32), jnp.float32)]
  scratch += [pltpu.SemaphoreType.DMA] * (3 * DEPTH)

  @functools.partial(pl.kernel, out_type=tuple(out_type), mesh=mesh,
                     scratch_types=tuple(scratch),
                     compiler_params=pltpu.CompilerParams(
                         needs_layout_passes=False,
                         use_tc_tiling_on_sc=False))
  def seg(*refs):
    it = iter(refs)
    z_hbm, e3_hbm = next(it), next(it)
    acc_out = next(it)
    deg_out = next(it) if do_deg else None
    rowsb, e3b, sidxb, gidxb = (next(it) for _ in range(4))
    if do_deg:
      degpriv = next(it)
    acc = next(it)
    semE = [next(it) for _ in range(DEPTH)]
    semG = [next(it) for _ in range(DEPTH)]
    semC = [next(it) for _ in range(DEPTH)]

    c = lax.axis_index("c")
    s = lax.axis_index("s")
    zeros16 = jnp.zeros((16,), jnp.float32)
    ones16 = jnp.ones((16,), jnp.float32)
    nb = NB_BASE + jnp.where(s < NB_EXTRA, 1, 0)
    b0 = s * NB_BASE + jnp.minimum(s, NB_EXTRA)
    row0 = s * ROWS_PER_TILE
    HALF = M // 2
    dbase = c * HALF

    if do_deg:
      def zdeg(i, _):
        degpriv[pl.ds(i * 16, 16)] = zeros16
        return 0
      lax.fori_loop(0, HALF // 16, zdeg, 0)

    def load_edges(b, hs, u):
      return pltpu.async_copy(e3_hbm.at[b0 + b], e3b.at[hs, u], semE[u])

    def prep_and_fire(hs, u, q, deg_pass):
      for j2 in range(K // 16):
        sl = pl.ds(j2 * 16, 16)
        sidx16 = e3b[hs, u, 2, sl] * NPAD + e3b[hs, u, 1, sl]
        sidxb[hs, u, sl] = sidx16
        gidxb[u, sl] = e3b[hs, u, 0, sl] + q * N
        if deg_pass:
          inr = (sidx16 >= dbase) & (sidx16 < dbase + HALF)
          plsc.addupdate_scatter(degpriv, [sidx16 - dbase], ones16, mask=inr)
      return pltpu.async_copy(z_hbm.at[gidxb.at[u]], rowsb.at[u], semG[u])

    def drain_scatter(hs, u):
      pltpu.make_async_copy(rowsb.at[u], acc.at[sidxb.at[hs, u]],
                            semC[u]).wait()

    for j in range(per_sc):
      q = c * per_sc + j
      deg_pass = do_deg and j == 0
      def zrows(i, _):
        rowsb[0, i, pl.ds(0, 16)] = zeros16
        rowsb[0, i, pl.ds(16, 16)] = zeros16
        return 0
      lax.fori_loop(0, K, zrows, 0)
      def zacc(k, _):
        pltpu.sync_copy(rowsb.at[0], acc.at[pl.ds(row0 + k * K, K)])
        return 0
      lax.fori_loop(0, ROWS_PER_TILE // K, zacc, 0)
      plsc.subcore_barrier()

      ngroups = nb // DEPTH
      for u in range(DEPTH):
        load_edges(u, 0, u)

      def group(g, _):
        hs = g & 1
        base = g * DEPTH
        gcp = []
        for u in range(DEPTH):
          pltpu.make_async_copy(e3_hbm.at[b0], e3b.at[hs, u], semE[u]).wait()
          @pl.when(g > 0)
          def _():
            drain_scatter(1 - hs, u)
          gcp.append(prep_and_fire(hs, u, q, deg_pass))
        @pl.when(g + 1 < ngroups)
        def _():
          for u in range(DEPTH):
            load_edges(base + DEPTH + u, 1 - hs, u)
        for u in range(DEPTH):
          gcp[u].wait()
          pltpu.async_copy(rowsb.at[u], acc.at[sidxb.at[hs, u]], semC[u],
                           add=True)
        return 0

      lax.fori_loop(0, ngroups, group, 0)
      for u in range(DEPTH):
        drain_scatter((ngroups - 1) & 1, u)

      def tail(b, _):
        load_edges(b, 0, 0).wait()
        prep_and_fire(0, 0, q, deg_pass).wait()
        pltpu.sync_copy(rowsb.at[0], acc.at[sidxb.at[0, 0]], add=True)
        return 0

      lax.fori_loop(ngroups * DEPTH, nb, tail, 0)
      if deg_pass:
        pltpu.sync_copy(degpriv, deg_out.at[c * 16 + s])
      plsc.subcore_barrier()
      pltpu.sync_copy(acc.at[pl.ds(row0, ROWS_PER_TILE)],
                      acc_out.at[pl.ds(row0, ROWS_PER_TILE),
                                 pl.ds(q * 32, 32)])
      plsc.subcore_barrier()

  return seg


EPT = E // 32
TRASH = 512
ACC2 = 544
G2 = 64


def _make_s2pruned():
  mesh = plsc.VectorSubcoreMesh(core_axis_name="c", subcore_axis_name="s")
  out_type = (jax.ShapeDtypeStruct((2, ACC2, 256), jnp.float32),
              jax.ShapeDtypeStruct((128,), jnp.int32))
  scratch = (
      pltpu.VMEM((N,), jnp.int32),
      pltpu.VMEM((128,), jnp.int32),
      pltpu.VMEM((128,), jnp.int32),
      pltpu.VMEM((EPT,), jnp.int32),
      pltpu.VMEM((EPT,), jnp.int32),
      pltpu.VMEM((EPT,), jnp.int32),
      pltpu.VMEM((160, G2), jnp.int32),
      pltpu.VMEM((160, G2), jnp.int32),
      pltpu.VMEM((2, G2, 256), jnp.float32),
      pltpu.VMEM_SHARED((ACC2, 256), jnp.float32),
      pltpu.SemaphoreType.DMA,
      pltpu.SemaphoreType.DMA,
  )

  @functools.partial(pl.kernel, out_type=out_type, mesh=mesh,
                     scratch_types=scratch,
                     compiler_params=pltpu.CompilerParams(
                         needs_layout_passes=False,
                         use_tc_tiling_on_sc=False))
  def s2k(h_hbm, src_hbm, dst_hbm, et_hbm, ids_hbm, accs_out, spos_out,
          slotmap, idsb, sposb, srcbig, dstbig, etbig, srcc, sidxc, rowsb,
          acc, semA, semB):
    c = lax.axis_index("c")
    s = lax.axis_index("s")
    w = c * 16 + s
    estart = w * EPT
    zeros16 = jnp.zeros((16,), jnp.float32)
    neg16 = jnp.full((16,), -1, jnp.int32)
    arange16 = jax.lax.iota(jnp.int32, 16)

    pltpu.sync_copy(ids_hbm, idsb)
    pltpu.sync_copy(src_hbm.at[pl.ds(estart, EPT)], srcbig)
    pltpu.sync_copy(dst_hbm.at[pl.ds(estart, EPT)], dstbig)
    pltpu.sync_copy(et_hbm.at[pl.ds(estart, EPT)], etbig)

    def fneg(i, _):
      slotmap[pl.ds(i * 16, 16)] = neg16
      return 0
    lax.fori_loop(0, N // 16, fneg, 0)
    for i in range(8):
      iv = idsb[pl.ds(i * 16, 16)]
      plsc.store_scatter(slotmap, [iv], arange16 + i * 16)

    def fpre(i, _):
      r, o = i // 4, (i % 4) * 16
      srcc[r, pl.ds(o, 16)] = jnp.zeros((16,), jnp.int32)
      sidxc[r, pl.ds(o, 16)] = jnp.full((16,), TRASH, jnp.int32)
      return 0
    lax.fori_loop(0, 160 * 4, fpre, 0)

    def zrows(i, _):
      def zr16(k, _):
        rowsb[0, i, pl.ds(k * 16, 16)] = zeros16
        return 0
      lax.fori_loop(0, 16, zr16, 0)
      return 0
    lax.fori_loop(0, 34, zrows, 0)
    pltpu.sync_copy(rowsb.at[0, pl.ds(0, 34)], acc.at[pl.ds(s * 34, 34)])
    plsc.subcore_barrier()

    def scan(i, off):
      sl = pl.ds(i * 16, 16)
      dv = dstbig[sl]
      slot16 = plsc.load_gather(slotmap, [dv])
      m = slot16 >= 0
      cs = plsc.cumsum(m.astype(jnp.int32))
      pos16 = off + cs - 1
      rowi = pos16 >> 6
      coli = pos16 & 63
      plsc.store_scatter(srcc, [rowi, coli], srcbig[sl], mask=m)
      plsc.store_scatter(sidxc, [rowi, coli], etbig[sl] * 128 + slot16,
                         mask=m)
      return off + jnp.max(cs)

    off = lax.fori_loop(0, EPT // 16, scan, jnp.int32(0))

    def pair(g, _):
      cpa = pltpu.async_copy(h_hbm.at[srcc.at[2 * g]], rowsb.at[0], semA)
      cpb = pltpu.async_copy(h_hbm.at[srcc.at[2 * g + 1]], rowsb.at[1], semB)
      cpa.wait()
      pltpu.sync_copy(rowsb.at[0], acc.at[sidxc.at[2 * g]], add=True)
      cpb.wait()
      pltpu.sync_copy(rowsb.at[1], acc.at[sidxc.at[2 * g + 1]], add=True)
      return 0

    lax.fori_loop(0, (off + 2 * G2 - 1) // (2 * G2), pair, 0)
    plsc.subcore_barrier()
    pltpu.sync_copy(acc.at[pl.ds(s * 34, 34)],
                    accs_out.at[c, pl.ds(s * 34, 34)])

    @pl.when(w == 0)
    def _():
      for i in range(8):
        sl = pl.ds(i * 16, 16)
        sposb[sl] = plsc.load_gather(slotmap, [idsb[sl]])
      pltpu.sync_copy(sposb, spos_out)

  return s2k


def _tc1_body(x_r, s1_r, deg_r, wroot_r, wrel_r, b_r, o_r):
  acc = jnp.dot(x_r[...], wroot_r[...], preferred_element_type=jnp.float32)
  acc = acc + b_r[...]
  degv = deg_r[...]
  s1 = s1_r[...]
  wrel = wrel_r[...]
  for r in range(NUM_REL):
    deg = jnp.sum(degv[:, r * 16:(r + 1) * 16], axis=1, keepdims=True)
    ir = 1.0 / jnp.clip(deg, 1.0, None)
    acc = acc + jnp.dot(s1[r] * ir, wrel[r],
                        preferred_element_type=jnp.float32)
  o_r[...] = jnp.maximum(acc, 0.0)


def _tc1(x, s1, deg, wroot, wrel, b):
  BN = 400
  grid = (N // BN,)
  return pl.pallas_call(
      _tc1_body,
      grid=grid,
      in_specs=[
          pl.BlockSpec((BN, 128), lambda i: (i, 0)),
          pl.BlockSpec((4, BN, 128), lambda i: (0, i, 0)),
          pl.BlockSpec((BN, 64), lambda i: (i, 0)),
          pl.BlockSpec((128, 256), lambda i: (0, 0)),
          pl.BlockSpec((4, 128, 256), lambda i: (0, 0, 0)),
          pl.BlockSpec((1, 256), lambda i: (0, 0)),
      ],
      out_specs=pl.BlockSpec((BN, 256), lambda i: (i, 0)),
      out_shape=jax.ShapeDtypeStruct((N, 256), jnp.float32),
  )(x, s1, deg, wroot, wrel, b)


CN = 1000


def _tc2_body(ids_r, h_r, deg_r, accs_r, spos_r, wroot_r, wrel_r, b2_r,
              fcw_r, fcb_r, o_r, hp_r, dp_r):
  i = pl.program_id(0)
  ids_v = ids_r[...]
  onehot = (lax.broadcasted_iota(jnp.int32, (128, CN), 1) + i * CN
            == ids_v).astype(jnp.float32)

  @pl.when(i == 0)
  def _():
    hp_r[...] = jnp.zeros_like(hp_r)
    dp_r[...] = jnp.zeros_like(dp_r)

  hp_r[...] += jnp.dot(onehot, h_r[...], preferred_element_type=jnp.float32)
  dp_r[...] += jnp.dot(onehot, deg_r[...], preferred_element_type=jnp.float32)

  @pl.when(i == N // CN - 1)
  def _():
    emb = jnp.dot(hp_r[...], wroot_r[...], preferred_element_type=jnp.float32)
    emb = emb + b2_r[...]
    dp = dp_r[...]
    accs = accs_r[...]
    s2sum = accs[0] + accs[1]
    oh_slot = (lax.broadcasted_iota(jnp.int32, (128, 128), 1)
               == spos_r[...]).astype(jnp.float32)
    wrel = wrel_r[...]
    for r in range(NUM_REL):
      s2p = jnp.dot(oh_slot, s2sum[r * 128:(r + 1) * 128],
                    preferred_element_type=jnp.float32)
      deg = jnp.sum(dp[:, r * 16:(r + 1) * 16], axis=1, keepdims=True)
      ir = 1.0 / jnp.clip(deg, 1.0, None)
      emb = emb + jnp.dot(s2p * ir, wrel[r],
                          preferred_element_type=jnp.float32)
    cat = jnp.concatenate([emb[0:64], emb[64:128]], axis=1)
    out = jnp.dot(cat, fcw_r[...], preferred_element_type=jnp.float32)
    o_r[...] = jnp.tanh(out + fcb_r[...])


def _tc2(ids, h, deg, accs, spos, wroot2, wrel2, b2, fcw, fcb):
  grid = (N // CN,)
  wob = pl.BlockSpec((128, 1), lambda i: (0, 0))
  outs = pl.pallas_call(
      _tc2_body,
      grid=grid,
      in_specs=[
          pl.BlockSpec((128, 1), lambda i: (0, 0)),
          pl.BlockSpec((CN, 256), lambda i: (i, 0)),
          pl.BlockSpec((CN, 64), lambda i: (i, 0)),
          pl.BlockSpec((2, ACC2, 256), lambda i: (0, 0, 0)),
          wob,
          pl.BlockSpec((256, 128), lambda i: (0, 0)),
          pl.BlockSpec((4, 256, 128), lambda i: (0, 0, 0)),
          pl.BlockSpec((1, 128), lambda i: (0, 0)),
          pl.BlockSpec((256, 256), lambda i: (0, 0)),
          pl.BlockSpec((1, 256), lambda i: (0, 0)),
      ],
      out_specs=[
          pl.BlockSpec((64, 256), lambda i: (0, 0)),
          pl.BlockSpec((128, 256), lambda i: (0, 0)),
          pl.BlockSpec((128, 64), lambda i: (0, 0)),
      ],
      out_shape=[
          jax.ShapeDtypeStruct((64, 256), jnp.float32),
          jax.ShapeDtypeStruct((128, 256), jnp.float32),
          jax.ShapeDtypeStruct((128, 64), jnp.float32),
      ],
  )(ids, h, deg, accs, spos, wroot2, wrel2, b2, fcw, fcb)
  return outs[0]


def kernel(x, edge_attr, W_rel1, W_root1, b1, W_rel2, W_root2, b2, fc_W,
           fc_b, edge_index, edge_type, nest_id, food_id):
  src = edge_index[0]
  dst = edge_index[1]
  e3 = jnp.stack([src.reshape(E // K, K), dst.reshape(E // K, K),
                  edge_type.reshape(E // K, K)], axis=1)

  xq = x.reshape(N, 4, 32).transpose(1, 0, 2).reshape(4 * N, 32)
  s1_m, deg32 = _make_segsum(4, True)(xq, e3)
  s1 = s1_m.reshape(NUM_REL, NPAD, 128)
  deg64 = (deg32.reshape(2, 16, 2, NPAD).transpose(3, 0, 2, 1)
           .reshape(NPAD, 64))

  h = _tc1(x, s1, deg64, W_root1, W_rel1, b1.reshape(1, 256))

  ids = jnp.concatenate([nest_id, food_id])
  accs, spos = _make_s2pruned()(h, src, dst, edge_type, ids)

  h0 = _tc2(ids.reshape(128, 1), h, deg64[:N], accs, spos.reshape(128, 1),
            W_root2, W_rel2, b2.reshape(1, 128), fc_W, fc_b.reshape(1, 256))
  return h0


# speedup vs baseline: 1.0207x; 1.0207x over previous
import functools

import jax
import jax.numpy as jnp
from jax import lax
from jax.experimental import pallas as pl
from jax.experimental.pallas import tpu as pltpu
from jax.experimental.pallas import tpu_sc as plsc

N = 10000
E = 320000
NUM_REL = 4
NPAD = 10240
M = NUM_REL * NPAD
K = 128
ROWS_PER_TILE = M // 16
NB_BASE = (E // K) // 16
NB_EXTRA = (E // K) % 16
NBMAX = NB_BASE + 1
EPAD = NBMAX * K * 16
DEPTH = 4


def _make_segsum(nq: int, do_deg: bool):
  per_sc = nq // 2
  mesh = plsc.VectorSubcoreMesh(core_axis_name="c", subcore_axis_name="s")
  out_type = [jax.ShapeDtypeStruct((M, nq * 32), jnp.float32)]
  if do_deg:
    out_type.append(jax.ShapeDtypeStruct((32, M // 2), jnp.float32))
  scratch = [
      pltpu.VMEM((DEPTH, K, 32), jnp.float32),
      pltpu.VMEM((2, DEPTH, 3, K), jnp.int32),
      pltpu.VMEM((2, DEPTH, K), jnp.int32),
      pltpu.VMEM((DEPTH, K), jnp.int32),
  ]
  if do_deg:
    scratch += [pltpu.VMEM((M // 2,), jnp.float32)]
  scratch += [pltpu.VMEM_SHARED((M, 32), jnp.float32)]
  scratch += [pltpu.SemaphoreType.DMA] * (3 * DEPTH)

  @functools.partial(pl.kernel, out_type=tuple(out_type), mesh=mesh,
                     scratch_types=tuple(scratch),
                     compiler_params=pltpu.CompilerParams(
                         needs_layout_passes=False,
                         use_tc_tiling_on_sc=False))
  def seg(*refs):
    it = iter(refs)
    z_hbm, e3_hbm = next(it), next(it)
    acc_out = next(it)
    deg_out = next(it) if do_deg else None
    rowsb, e3b, sidxb, gidxb = (next(it) for _ in range(4))
    if do_deg:
      degpriv = next(it)
    acc = next(it)
    semE = [next(it) for _ in range(DEPTH)]
    semG = [next(it) for _ in range(DEPTH)]
    semC = [next(it) for _ in range(DEPTH)]

    c = lax.axis_index("c")
    s = lax.axis_index("s")
    zeros16 = jnp.zeros((16,), jnp.float32)
    ones16 = jnp.ones((16,), jnp.float32)
    nb = NB_BASE + jnp.where(s < NB_EXTRA, 1, 0)
    b0 = s * NB_BASE + jnp.minimum(s, NB_EXTRA)
    row0 = s * ROWS_PER_TILE
    HALF = M // 2
    dbase = c * HALF

    if do_deg:
      def zdeg(i, _):
        degpriv[pl.ds(i * 16, 16)] = zeros16
        return 0
      lax.fori_loop(0, HALF // 16, zdeg, 0)

    def load_edges(b, hs, u):
      return pltpu.async_copy(e3_hbm.at[b0 + b], e3b.at[hs, u], semE[u])

    def prep_and_fire(hs, u, q, deg_pass):
      for j2 in range(K // 16):
        sl = pl.ds(j2 * 16, 16)
        sidx16 = e3b[hs, u, 2, sl] * NPAD + e3b[hs, u, 1, sl]
        sidxb[hs, u, sl] = sidx16
        gidxb[u, sl] = e3b[hs, u, 0, sl] + q * N
        if deg_pass:
          inr = (sidx16 >= dbase) & (sidx16 < dbase + HALF)
          plsc.addupdate_scatter(degpriv, [sidx16 - dbase], ones16, mask=inr)
      return pltpu.async_copy(z_hbm.at[gidxb.at[u]], rowsb.at[u], semG[u])

    def drain_scatter(hs, u):
      pltpu.make_async_copy(rowsb.at[u], acc.at[sidxb.at[hs, u]],
                            semC[u]).wait()

    for j in range(per_sc):
      q = c * per_sc + j
      deg_pass = do_deg and j == 0
      def zrows(i, _):
        rowsb[0, i, pl.ds(0, 16)] = zeros16
        rowsb[0, i, pl.ds(16, 16)] = zeros16
        return 0
      lax.fori_loop(0, K, zrows, 0)
      def zacc(k, _):
        pltpu.sync_copy(rowsb.at[0], acc.at[pl.ds(row0 + k * K, K)])
        return 0
      lax.fori_loop(0, ROWS_PER_TILE // K, zacc, 0)
      plsc.subcore_barrier()

      ngroups = nb // DEPTH
      for u in range(DEPTH):
        load_edges(u, 0, u)

      def group(g, _):
        hs = g & 1
        base = g * DEPTH
        gcp = []
        for u in range(DEPTH):
          pltpu.make_async_copy(e3_hbm.at[b0], e3b.at[hs, u], semE[u]).wait()
          @pl.when(g > 0)
          def _():
            drain_scatter(1 - hs, u)
          gcp.append(prep_and_fire(hs, u, q, deg_pass))
        @pl.when(g + 1 < ngroups)
        def _():
          for u in range(DEPTH):
            load_edges(base + DEPTH + u, 1 - hs, u)
        for u in range(DEPTH):
          gcp[u].wait()
          pltpu.async_copy(rowsb.at[u], acc.at[sidxb.at[hs, u]], semC[u],
                           add=True)
        return 0

      lax.fori_loop(0, ngroups, group, 0)
      for u in range(DEPTH):
        drain_scatter((ngroups - 1) & 1, u)

      def tail(b, _):
        load_edges(b, 0, 0).wait()
        prep_and_fire(0, 0, q, deg_pass).wait()
        pltpu.sync_copy(rowsb.at[0], acc.at[sidxb.at[0, 0]], add=True)
        return 0

      lax.fori_loop(ngroups * DEPTH, nb, tail, 0)
      if deg_pass:
        pltpu.sync_copy(degpriv, deg_out.at[c * 16 + s])
      plsc.subcore_barrier()
      pltpu.sync_copy(acc.at[pl.ds(row0, ROWS_PER_TILE)],
                      acc_out.at[pl.ds(row0, ROWS_PER_TILE),
                                 pl.ds(q * 32, 32)])
      plsc.subcore_barrier()

  return seg


EPT = E // 32
TRASH = 512
ACC2 = 544
G2 = 64


def _make_s2pruned():
  mesh = plsc.VectorSubcoreMesh(core_axis_name="c", subcore_axis_name="s")
  out_type = (jax.ShapeDtypeStruct((2, ACC2, 256), jnp.float32),
              jax.ShapeDtypeStruct((128,), jnp.int32))
  scratch = (
      pltpu.VMEM((N,), jnp.int32),
      pltpu.VMEM((128,), jnp.int32),
      pltpu.VMEM((128,), jnp.int32),
      pltpu.VMEM((EPT,), jnp.int32),
      pltpu.VMEM((EPT,), jnp.int32),
      pltpu.VMEM((EPT,), jnp.int32),
      pltpu.VMEM((160, G2), jnp.int32),
      pltpu.VMEM((160, G2), jnp.int32),
      pltpu.VMEM((2, G2, 256), jnp.float32),
      pltpu.VMEM_SHARED((ACC2, 256), jnp.float32),
      pltpu.SemaphoreType.DMA,
      pltpu.SemaphoreType.DMA,
  )

  @functools.partial(pl.kernel, out_type=out_type, mesh=mesh,
                     scratch_types=scratch,
                     compiler_params=pltpu.CompilerParams(
                         needs_layout_passes=False,
                         use_tc_tiling_on_sc=False))
  def s2k(h_hbm, src_hbm, dst_hbm, et_hbm, ids_hbm, accs_out, spos_out,
          slotmap, idsb, sposb, srcbig, dstbig, etbig, srcc, sidxc, rowsb,
          acc, semA, semB):
    c = lax.axis_index("c")
    s = lax.axis_index("s")
    w = c * 16 + s
    estart = w * EPT
    zeros16 = jnp.zeros((16,), jnp.float32)
    neg16 = jnp.full((16,), -1, jnp.int32)
    arange16 = jax.lax.iota(jnp.int32, 16)

    pltpu.sync_copy(ids_hbm, idsb)
    pltpu.sync_copy(src_hbm.at[pl.ds(estart, EPT)], srcbig)
    pltpu.sync_copy(dst_hbm.at[pl.ds(estart, EPT)], dstbig)
    pltpu.sync_copy(et_hbm.at[pl.ds(estart, EPT)], etbig)

    def fneg(i, _):
      slotmap[pl.ds(i * 16, 16)] = neg16
      return 0
    lax.fori_loop(0, N // 16, fneg, 0)
    for i in range(8):
      iv = idsb[pl.ds(i * 16, 16)]
      plsc.store_scatter(slotmap, [iv], arange16 + i * 16)

    def fpre(i, _):
      r, o = i // 4, (i % 4) * 16
      srcc[r, pl.ds(o, 16)] = jnp.zeros((16,), jnp.int32)
      sidxc[r, pl.ds(o, 16)] = jnp.full((16,), TRASH, jnp.int32)
      return 0
    lax.fori_loop(0, 160 * 4, fpre, 0)

    def zrows(i, _):
      def zr16(k, _):
        rowsb[0, i, pl.ds(k * 16, 16)] = zeros16
        return 0
      lax.fori_loop(0, 16, zr16, 0)
      return 0
    lax.fori_loop(0, 34, zrows, 0)
    pltpu.sync_copy(rowsb.at[0, pl.ds(0, 34)], acc.at[pl.ds(s * 34, 34)])
    plsc.subcore_barrier()

    def scan(i, off):
      sl = pl.ds(i * 16, 16)
      dv = dstbig[sl]
      slot16 = plsc.load_gather(slotmap, [dv])
      m = slot16 >= 0
      cs = plsc.cumsum(m.astype(jnp.int32))
      pos16 = off + cs - 1
      rowi = pos16 >> 6
      coli = pos16 & 63
      plsc.store_scatter(srcc, [rowi, coli], srcbig[sl], mask=m)
      plsc.store_scatter(sidxc, [rowi, coli], etbig[sl] * 128 + slot16,
                         mask=m)
      return off + jnp.max(cs)

    off = lax.fori_loop(0, EPT // 16, scan, jnp.int32(0))

    def pair(g, _):
      cpa = pltpu.async_copy(h_hbm.at[srcc.at[2 * g]], rowsb.at[0], semA)
      cpb = pltpu.async_copy(h_hbm.at[srcc.at[2 * g + 1]], rowsb.at[1], semB)
      cpa.wait()
      pltpu.sync_copy(rowsb.at[0], acc.at[sidxc.at[2 * g]], add=True)
      cpb.wait()
      pltpu.sync_copy(rowsb.at[1], acc.at[sidxc.at[2 * g + 1]], add=True)
      return 0

    lax.fori_loop(0, (off + 2 * G2 - 1) // (2 * G2), pair, 0)
    plsc.subcore_barrier()
    pltpu.sync_copy(acc.at[pl.ds(s * 34, 34)],
                    accs_out.at[c, pl.ds(s * 34, 34)])

    @pl.when(w == 0)
    def _():
      for i in range(8):
        sl = pl.ds(i * 16, 16)
        sposb[sl] = plsc.load_gather(slotmap, [idsb[sl]])
      pltpu.sync_copy(sposb, spos_out)

  return s2k


def _tc1_body(x_r, s1_r, deg_r, wroot_r, wrel_r, b_r, o_r):
  acc = jnp.dot(x_r[...], wroot_r[...], preferred_element_type=jnp.float32)
  acc = acc + b_r[...]
  degv = deg_r[...]
  s1 = s1_r[...]
  wrel = wrel_r[...]
  for r in range(NUM_REL):
    deg = jnp.sum(degv[:, r * 16:(r + 1) * 16], axis=1, keepdims=True)
    ir = 1.0 / jnp.clip(deg, 1.0, None)
    acc = acc + jnp.dot(s1[r] * ir, wrel[r],
                        preferred_element_type=jnp.float32)
  o_r[...] = jnp.maximum(acc, 0.0)


def _tc1(x, s1, deg, wroot, wrel, b):
  BN = 400
  grid = (N // BN,)
  return pl.pallas_call(
      _tc1_body,
      grid=grid,
      in_specs=[
          pl.BlockSpec((BN, 128), lambda i: (i, 0)),
          pl.BlockSpec((4, BN, 128), lambda i: (0, i, 0)),
          pl.BlockSpec((BN, 64), lambda i: (i, 0)),
          pl.BlockSpec((128, 256), lambda i: (0, 0)),
          pl.BlockSpec((4, 128, 256), lambda i: (0, 0, 0)),
          pl.BlockSpec((1, 256), lambda i: (0, 0)),
      ],
      out_specs=pl.BlockSpec((BN, 256), lambda i: (i, 0)),
      out_shape=jax.ShapeDtypeStruct((N, 256), jnp.float32),
  )(x, s1, deg, wroot, wrel, b)


CN = 1000


def _tc2_body(ids_r, h_r, deg_r, accs_r, spos_r, wroot_r, wrel_r, b2_r,
              fcw_r, fcb_r, o_r, hp_r, dp_r):
  i = pl.program_id(0)
  ids_v = ids_r[...]
  onehot = (lax.broadcasted_iota(jnp.int32, (128, CN), 1) + i * CN
            == ids_v).astype(jnp.float32)

  @pl.when(i == 0)
  def _():
    hp_r[...] = jnp.zeros_like(hp_r)
    dp_r[...] = jnp.zeros_like(dp_r)

  hp_r[...] += jnp.dot(onehot, h_r[...], preferred_element_type=jnp.float32)
  dp_r[...] += jnp.dot(onehot, deg_r[...], preferred_element_type=jnp.float32)

  @pl.when(i == N // CN - 1)
  def _():
    emb = jnp.dot(hp_r[...], wroot_r[...], preferred_element_type=jnp.float32)
    emb = emb + b2_r[...]
    dp = dp_r[...]
    accs = accs_r[...]
    s2sum = accs[0] + accs[1]
    oh_slot = (lax.broadcasted_iota(jnp.int32, (128, 128), 1)
               == spos_r[...]).astype(jnp.float32)
    wrel = wrel_r[...]
    for r in range(NUM_REL):
      s2p = jnp.dot(oh_slot, s2sum[r * 128:(r + 1) * 128],
                    preferred_element_type=jnp.float32)
      deg = jnp.sum(dp[:, r * 16:(r + 1) * 16], axis=1, keepdims=True)
      ir = 1.0 / jnp.clip(deg, 1.0, None)
      emb = emb + jnp.dot(s2p * ir, wrel[r],
                          preferred_element_type=jnp.float32)
    cat = jnp.concatenate([emb[0:64], emb[64:128]], axis=1)
    out = jnp.dot(cat, fcw_r[...], preferred_element_type=jnp.float32)
    o_r[...] = jnp.tanh(out + fcb_r[...])


def _tc2(ids, h, deg, accs, spos, wroot2, wrel2, b2, fcw, fcb):
  grid = (N // CN,)
  wob = pl.BlockSpec((128, 1), lambda i: (0, 0))
  outs = pl.pallas_call(
      _tc2_body,
      grid=grid,
      in_specs=[
          pl.BlockSpec((128, 1), lambda i: (0, 0)),
          pl.BlockSpec((CN, 256), lambda i: (i, 0)),
          pl.BlockSpec((CN, 64), lambda i: (i, 0)),
          pl.BlockSpec((2, ACC2, 256), lambda i: (0, 0, 0)),
          wob,
          pl.BlockSpec((256, 128), lambda i: (0, 0)),
          pl.BlockSpec((4, 256, 128), lambda i: (0, 0, 0)),
          pl.BlockSpec((1, 128), lambda i: (0, 0)),
          pl.BlockSpec((256, 256), lambda i: (0, 0)),
          pl.BlockSpec((1, 256), lambda i: (0, 0)),
      ],
      out_specs=[
          pl.BlockSpec((64, 256), lambda i: (0, 0)),
          pl.BlockSpec((128, 256), lambda i: (0, 0)),
          pl.BlockSpec((128, 64), lambda i: (0, 0)),
      ],
      out_shape=[
          jax.ShapeDtypeStruct((64, 256), jnp.float32),
          jax.ShapeDtypeStruct((128, 256), jnp.float32),
          jax.ShapeDtypeStruct((128, 64), jnp.float32),
      ],
  )(ids, h, deg, accs, spos, wroot2, wrel2, b2, fcw, fcb)
  return outs[0]


def kernel(x, edge_attr, W_rel1, W_root1, b1, W_rel2, W_root2, b2, fc_W,
           fc_b, edge_index, edge_type, nest_id, food_id):
  src = edge_index[0]
  dst = edge_index[1]
  e3 = jnp.stack([src.reshape(E // K, K), dst.reshape(E // K, K),
                  edge_type.reshape(E // K, K)], axis=1)

  xq = x.reshape(N, 4, 32).transpose(1, 0, 2).reshape(4 * N, 32)
  s1_m, deg32 = _make_segsum(4, True)(xq, e3)
  s1 = s1_m.reshape(NUM_REL, NPAD, 128)
  deg64 = (deg32.reshape(2, 16, 2, NPAD).transpose(3, 0, 2, 1)
           .reshape(NPAD, 64))

  h = _tc1(x, s1, deg64, W_root1, W_rel1, b1.reshape(1, 256))

  ids = jnp.concatenate([nest_id, food_id])
  accs, spos = _make_s2pruned()(h, src, dst, edge_type, ids)

  h0 = _tc2(ids.reshape(128, 1), h, deg64[:N], accs, spos.reshape(128, 1),
            W_root2, W_rel2, b2.reshape(1, 128), fc_W, fc_b.reshape(1, 256))
  return h0
